# baseline (speedup 1.0000x reference)
import functools

import jax
import jax.numpy as jnp
from jax import lax
from jax.experimental import pallas as pl
from jax.experimental.pallas import tpu as pltpu
from jax.experimental.pallas import tpu_sc as plsc

N0 = 50000
N1 = 10000
B = 2048
E0 = 160000
E1 = 32768
F = 256
HID = 256
NCLS = 40

NS = 16
NC = 2
CH = 128
DEPTH = 2


IB = 16


def _make_seg_kernel(nblk, zero_pt, out_pt):
    nch = nblk * IB
    acc_rows = NS * zero_pt
    out_rows = NS * out_pt

    mesh = plsc.VectorSubcoreMesh(core_axis_name="c", subcore_axis_name="s",
                                  num_cores=NC, num_subcores=NS)

    @functools.partial(
        pl.kernel,
        mesh=mesh,
        out_type=[
            jax.ShapeDtypeStruct((NC, out_rows, 128), jnp.float32),
        ],
        scratch_types=[
            pltpu.VMEM((IB, CH), jnp.int32),
            pltpu.VMEM((IB, CH), jnp.int32),
        ] + [pltpu.VMEM((CH, 128), jnp.float32)] * DEPTH + [
            pltpu.VMEM_SHARED((acc_rows, 128), jnp.float32),
        ] + [pltpu.SemaphoreType.DMA] * (2 * DEPTH),
    )
    def seg(x2, gsrc, dstr, zeros,
            agg_out,
            gidx_v, dst_v, *rest):
        bufs = rest[:DEPTH]
        acc_sh = rest[DEPTH]
        gsem = rest[DEPTH + 1:2 * DEPTH + 1]
        ssem = rest[2 * DEPTH + 1:]
        c = lax.axis_index("c")
        s = lax.axis_index("s")
        pltpu.sync_copy(zeros.at[pl.ds(0, zero_pt)],
                        acc_sh.at[pl.ds(s * zero_pt, zero_pt)])
        plsc.subcore_barrier()

        def blk(b, carry):
            pltpu.sync_copy(gsrc.at[c, s, pl.ds(b * IB, IB)], gidx_v)
            pltpu.sync_copy(dstr.at[s, pl.ds(b * IB, IB)], dst_v)
            gath = [None] * DEPTH
            scat = [None] * DEPTH
            for j in range(min(2, IB)):
                gath[j % DEPTH] = pltpu.async_copy(
                    x2.at[gidx_v.at[j]], bufs[j % DEPTH], gsem[j % DEPTH])
            for j in range(IB):
                cur = j % DEPTH
                gath[cur].wait()
                scat[cur] = pltpu.async_copy(bufs[cur],
                                             acc_sh.at[dst_v.at[j]],
                                             ssem[cur], add=True)
                m = j + 2
                if m < IB:
                    mb = m % DEPTH
                    if scat[mb] is not None:
                        scat[mb].wait()
                        scat[mb] = None
                    gath[mb] = pltpu.async_copy(x2.at[gidx_v.at[m]],
                                                bufs[mb], gsem[mb])
            for sc_ in scat:
                if sc_ is not None:
                    sc_.wait()
            return carry

        lax.fori_loop(0, nblk, blk, 0)
        plsc.subcore_barrier()
        pltpu.sync_copy(acc_sh.at[pl.ds(s * out_pt, out_pt)],
                        agg_out.at[c, pl.ds(s * out_pt, out_pt)])
    return seg


def _make_cnt_kernel():
    mesh = plsc.VectorSubcoreMesh(core_axis_name="c", subcore_axis_name="s",
                                  num_cores=NC, num_subcores=NS)
    r0 = NS * 632

    @functools.partial(
        pl.kernel,
        mesh=mesh,
        out_type=[
            jax.ShapeDtypeStruct((NC, r0, 128), jnp.float32),
            jax.ShapeDtypeStruct((NC, B, 128), jnp.float32),
        ],
        scratch_types=[
            pltpu.VMEM((IB, CH), jnp.int32),
            pltpu.VMEM((CH, 128), jnp.float32),
            pltpu.VMEM_SHARED((r0, 128), jnp.float32),
            pltpu.VMEM_SHARED((B, 128), jnp.float32),
            pltpu.SemaphoreType.DMA,
        ],
    )
    def cnt(dstr0, dstr1, ones, zeros, cnt0_out, cnt1_out,
            dst_v, ones_v, acc0_sh, acc1_sh, sem):
        c = lax.axis_index("c")
        s = lax.axis_index("s")
        pltpu.sync_copy(zeros.at[pl.ds(0, 632)],
                        acc0_sh.at[pl.ds(s * 632, 632)])
        pltpu.sync_copy(zeros.at[pl.ds(0, 128)],
                        acc1_sh.at[pl.ds(s * 128, 128)])
        pltpu.sync_copy(ones, ones_v)
        plsc.subcore_barrier()

        def count_into(dstr, acc_sh, nblk):
            def blk(b, carry):
                pltpu.sync_copy(dstr.at[s, pl.ds(b * IB, IB)], dst_v)
                cps = []
                for k in range(IB // 2):
                    cps.append(pltpu.async_copy(
                        ones_v, acc_sh.at[dst_v.at[2 * k + c]], sem,
                        add=True))
                for cp in cps:
                    cp.wait()
                return carry
            lax.fori_loop(0, nblk, blk, 0)

        count_into(dstr0, acc0_sh, 5)
        count_into(dstr1, acc1_sh, 1)
        plsc.subcore_barrier()
        pltpu.sync_copy(acc0_sh.at[pl.ds(s * 632, 632)],
                        cnt0_out.at[c, pl.ds(s * 632, 632)])
        pltpu.sync_copy(acc1_sh.at[pl.ds(s * 128, 128)],
                        cnt1_out.at[c, pl.ds(s * 128, 128)])

    return cnt


_E0P = NS * 5 * IB * CH
_seg_cache = {}


def _seg_kernels():
    if not _seg_cache:
        _seg_cache["l0"] = _make_seg_kernel(nblk=5, zero_pt=632, out_pt=632)
        _seg_cache["l1"] = _make_seg_kernel(nblk=1, zero_pt=128, out_pt=128)
        _seg_cache["cnt"] = _make_cnt_kernel()
    return _seg_cache["l0"], _seg_cache["l1"], _seg_cache["cnt"]


def _l0_body(agg_ref, cnt_ref, x_ref, wn_ref, ws_ref, b_ref, o_ref):
    cnt = jnp.maximum(cnt_ref[0, :, 0:1] + cnt_ref[1, :, 0:1], 1.0)
    m0 = agg_ref[0] / cnt
    m1 = agg_ref[1] / cnt
    h = (jnp.dot(m0, wn_ref[0:128, :], preferred_element_type=jnp.float32)
         + jnp.dot(m1, wn_ref[128:256, :], preferred_element_type=jnp.float32)
         + jnp.dot(x_ref[...], ws_ref[...], preferred_element_type=jnp.float32)
         + b_ref[...])
    o_ref[...] = jnp.maximum(h, 0.0)


def _l1_body(agg_ref, cnt_ref, h1_ref, wn_ref, ws_ref, b_ref, av_ref,
             wo_ref, bo_ref, o_ref, alpha_ref):
    cnt = jnp.maximum(cnt_ref[0, :, 0:1] + cnt_ref[1, :, 0:1], 1.0)
    m0 = agg_ref[0] / cnt
    m1 = agg_ref[1] / cnt
    h1b = h1_ref[...]
    h2 = (jnp.dot(m0, wn_ref[0:128, :], preferred_element_type=jnp.float32)
          + jnp.dot(m1, wn_ref[128:256, :], preferred_element_type=jnp.float32)
          + jnp.dot(h1b, ws_ref[...], preferred_element_type=jnp.float32)
          + b_ref[...])
    av = av_ref[...]
    s0 = jnp.sum(jnp.tanh(h1b) * av, axis=1, keepdims=True)
    s1 = jnp.sum(jnp.tanh(h2) * av, axis=1, keepdims=True)
    m = jnp.maximum(s0, s1)
    e0 = jnp.exp(s0 - m)
    e1 = jnp.exp(s1 - m)
    den = e0 + e1
    a0 = e0 / den
    a1 = e1 / den
    h = a0 * h1b + a1 * h2
    o_ref[...] = (jnp.dot(h, wo_ref[...], preferred_element_type=jnp.float32)
                  + bo_ref[...])
    alpha_ref[...] = jnp.concatenate([a0, a1], axis=1)


def _edge_prep(src, dst, e_pad, trash_row, nch):
    src = src.astype(jnp.int32)
    dst = dst.astype(jnp.int32)
    pad = e_pad - src.shape[0]
    if pad:
        src = jnp.concatenate([src, jnp.zeros((pad,), jnp.int32)])
        dst = jnp.concatenate([dst, jnp.full((pad,), trash_row, jnp.int32)])
    s2 = src * 2
    gsrc = jnp.stack([s2, s2 + 1]).reshape(NC, NS, nch, CH)
    dstr = dst.reshape(NS, nch, CH)
    return gsrc, dstr


def kernel(x, edge_index_0, edge_index_1, W_n0, W_s0, b0, W_n1, W_s1, b1,
           att_vec, W_out, b_out):
    f32 = jnp.float32
    ones = jnp.ones((CH, 128), f32)
    zeros = jnp.zeros((632, 128), f32)

    _seg0, _seg1, _cntk = _seg_kernels()
    gsrc0, dstr0 = _edge_prep(edge_index_0[0], edge_index_0[1], _E0P, N1,
                              5 * IB)
    gsrc1, dstr1 = _edge_prep(edge_index_1[0], edge_index_1[1], E1, B, IB)

    cnt0, cnt1 = _cntk(dstr0, dstr1, ones, zeros)
    x2 = x.reshape(2 * N0, 128)
    (agg0,) = _seg0(x2, gsrc0, dstr0, zeros)

    bm0 = 1000
    h1 = pl.pallas_call(
        _l0_body,
        grid=(N1 // bm0,),
        in_specs=[
            pl.BlockSpec((NC, bm0, 128), lambda i: (0, i, 0)),
            pl.BlockSpec((NC, bm0, 128), lambda i: (0, i, 0)),
            pl.BlockSpec((bm0, F), lambda i: (i, 0)),
            pl.BlockSpec((F, HID), lambda i: (0, 0)),
            pl.BlockSpec((F, HID), lambda i: (0, 0)),
            pl.BlockSpec((1, HID), lambda i: (0, 0)),
        ],
        out_specs=pl.BlockSpec((bm0, HID), lambda i: (i, 0)),
        out_shape=jax.ShapeDtypeStruct((N1, HID), f32),
    )(agg0, cnt0, x, W_n0, W_s0, b0.reshape(1, HID))

    h1_2 = h1.reshape(2 * N1, 128)
    (agg1,) = _seg1(h1_2, gsrc1, dstr1, zeros)

    bm1 = 512
    out, alpha = pl.pallas_call(
        _l1_body,
        grid=(B // bm1,),
        in_specs=[
            pl.BlockSpec((NC, bm1, 128), lambda i: (0, i, 0)),
            pl.BlockSpec((NC, bm1, 128), lambda i: (0, i, 0)),
            pl.BlockSpec((bm1, HID), lambda i: (i, 0)),
            pl.BlockSpec((HID, HID), lambda i: (0, 0)),
            pl.BlockSpec((HID, HID), lambda i: (0, 0)),
            pl.BlockSpec((1, HID), lambda i: (0, 0)),
            pl.BlockSpec((1, HID), lambda i: (0, 0)),
            pl.BlockSpec((HID, NCLS), lambda i: (0, 0)),
            pl.BlockSpec((1, NCLS), lambda i: (0, 0)),
        ],
        out_specs=[
            pl.BlockSpec((bm1, NCLS), lambda i: (i, 0)),
            pl.BlockSpec((bm1, 2), lambda i: (i, 0)),
        ],
        out_shape=[
            jax.ShapeDtypeStruct((B, NCLS), f32),
            jax.ShapeDtypeStruct((B, 2), f32),
        ],
    )(agg1, cnt1, h1, W_n1, W_s1, b1.reshape(1, HID),
      att_vec.reshape(1, HID), W_out, b_out.reshape(1, NCLS))

    return (out, alpha)


# speedup vs baseline: 1.1112x; 1.1112x over previous
import functools

import jax
import jax.numpy as jnp
from jax import lax
from jax.experimental import pallas as pl
from jax.experimental.pallas import tpu as pltpu
from jax.experimental.pallas import tpu_sc as plsc

N0 = 50000
N1 = 10000
B = 2048
E0 = 160000
E1 = 32768
F = 256
HID = 256
NCLS = 40

NS = 16
NC = 2
CH = 128
DEPTH = 2


IB = 16


def _make_seg_kernel(nblk, zero_pt, out_pt):
    nch = nblk * IB
    acc_rows = NS * zero_pt
    out_rows = NS * out_pt

    mesh = plsc.VectorSubcoreMesh(core_axis_name="c", subcore_axis_name="s",
                                  num_cores=NC, num_subcores=NS)

    @functools.partial(
        pl.kernel,
        mesh=mesh,
        out_type=[
            jax.ShapeDtypeStruct((NC, out_rows, 128), jnp.float32),
        ],
        scratch_types=[
            pltpu.VMEM((IB, CH), jnp.int32),
            pltpu.VMEM((IB, CH), jnp.int32),
        ] + [pltpu.VMEM((CH, 128), jnp.float32)] * DEPTH + [
            pltpu.VMEM_SHARED((acc_rows, 128), jnp.float32),
        ] + [pltpu.SemaphoreType.DMA] * (2 * DEPTH),
    )
    def seg(x2, gsrc, dstr, zeros,
            agg_out,
            gidx_v, dst_v, *rest):
        bufs = rest[:DEPTH]
        acc_sh = rest[DEPTH]
        gsem = rest[DEPTH + 1:2 * DEPTH + 1]
        ssem = rest[2 * DEPTH + 1:]
        c = lax.axis_index("c")
        s = lax.axis_index("s")
        pltpu.sync_copy(zeros.at[pl.ds(0, zero_pt)],
                        acc_sh.at[pl.ds(s * zero_pt, zero_pt)])
        plsc.subcore_barrier()

        def blk(b, carry):
            pltpu.sync_copy(gsrc.at[c, s, pl.ds(b * IB, IB)], gidx_v)
            pltpu.sync_copy(dstr.at[s, pl.ds(b * IB, IB)], dst_v)
            gath = [None] * DEPTH
            scat = [None] * DEPTH
            for j in range(min(2, IB)):
                gath[j % DEPTH] = pltpu.async_copy(
                    x2.at[gidx_v.at[j]], bufs[j % DEPTH], gsem[j % DEPTH])
            for j in range(IB):
                cur = j % DEPTH
                gath[cur].wait()
                scat[cur] = pltpu.async_copy(bufs[cur],
                                             acc_sh.at[dst_v.at[j]],
                                             ssem[cur], add=True)
                m = j + 2
                if m < IB:
                    mb = m % DEPTH
                    if scat[mb] is not None:
                        scat[mb].wait()
                        scat[mb] = None
                    gath[mb] = pltpu.async_copy(x2.at[gidx_v.at[m]],
                                                bufs[mb], gsem[mb])
            for sc_ in scat:
                if sc_ is not None:
                    sc_.wait()
            return carry

        lax.fori_loop(0, nblk, blk, 0)
        plsc.subcore_barrier()
        pltpu.sync_copy(acc_sh.at[pl.ds(s * out_pt, out_pt)],
                        agg_out.at[c, pl.ds(s * out_pt, out_pt)])
    return seg


def _make_cnt_kernel():
    mesh = plsc.VectorSubcoreMesh(core_axis_name="c", subcore_axis_name="s",
                                  num_cores=NC, num_subcores=NS)
    r0 = NS * 632

    @functools.partial(
        pl.kernel,
        mesh=mesh,
        out_type=[
            jax.ShapeDtypeStruct((NC, r0, 128), jnp.float32),
            jax.ShapeDtypeStruct((NC, B, 128), jnp.float32),
        ],
        scratch_types=[
            pltpu.VMEM((IB, CH), jnp.int32),
            pltpu.VMEM((CH, 128), jnp.float32),
            pltpu.VMEM_SHARED((r0, 128), jnp.float32),
            pltpu.VMEM_SHARED((B, 128), jnp.float32),
            pltpu.SemaphoreType.DMA,
        ],
    )
    def cnt(dstr0, dstr1, ones, zeros, cnt0_out, cnt1_out,
            dst_v, ones_v, acc0_sh, acc1_sh, sem):
        c = lax.axis_index("c")
        s = lax.axis_index("s")
        pltpu.sync_copy(zeros.at[pl.ds(0, 632)],
                        acc0_sh.at[pl.ds(s * 632, 632)])
        pltpu.sync_copy(zeros.at[pl.ds(0, 128)],
                        acc1_sh.at[pl.ds(s * 128, 128)])
        pltpu.sync_copy(ones, ones_v)
        plsc.subcore_barrier()

        def count_into(dstr, acc_sh, nblk):
            def blk(b, carry):
                pltpu.sync_copy(dstr.at[s, pl.ds(b * IB, IB)], dst_v)
                cps = []
                for k in range(IB // 2):
                    cps.append(pltpu.async_copy(
                        ones_v, acc_sh.at[dst_v.at[2 * k + c]], sem,
                        add=True))
                for cp in cps:
                    cp.wait()
                return carry
            lax.fori_loop(0, nblk, blk, 0)

        count_into(dstr0, acc0_sh, 5)
        count_into(dstr1, acc1_sh, 1)
        plsc.subcore_barrier()
        pltpu.sync_copy(acc0_sh.at[pl.ds(s * 632, 632)],
                        cnt0_out.at[c, pl.ds(s * 632, 632)])
        pltpu.sync_copy(acc1_sh.at[pl.ds(s * 128, 128)],
                        cnt1_out.at[c, pl.ds(s * 128, 128)])

    return cnt


_E0P = NS * 5 * IB * CH
_seg_cache = {}


def _seg_kernels():
    if not _seg_cache:
        _seg_cache["l0"] = _make_seg_kernel(nblk=5, zero_pt=632, out_pt=632)
        _seg_cache["l1"] = _make_seg_kernel(nblk=1, zero_pt=128, out_pt=128)
        _seg_cache["cnt"] = _make_cnt_kernel()
    return _seg_cache["l0"], _seg_cache["l1"], _seg_cache["cnt"]


def _l0_body(agg_ref, cnt_ref, x_ref, wn_ref, ws_ref, b_ref, o_ref):
    cnt = jnp.maximum(cnt_ref[0, :, 0:1] + cnt_ref[1, :, 0:1], 1.0)
    m0 = agg_ref[0] / cnt
    m1 = agg_ref[1] / cnt
    h = (jnp.dot(m0, wn_ref[0:128, :], preferred_element_type=jnp.float32)
         + jnp.dot(m1, wn_ref[128:256, :], preferred_element_type=jnp.float32)
         + jnp.dot(x_ref[...], ws_ref[...], preferred_element_type=jnp.float32)
         + b_ref[...])
    o_ref[...] = jnp.maximum(h, 0.0)


def _l1_body(agg_ref, cnt_ref, h1_ref, wn_ref, ws_ref, b_ref, av_ref,
             wo_ref, bo_ref, o_ref, alpha_ref):
    cnt = jnp.maximum(cnt_ref[0, :, 0:1] + cnt_ref[1, :, 0:1], 1.0)
    m0 = agg_ref[0] / cnt
    m1 = agg_ref[1] / cnt
    h1b = h1_ref[...]
    h2 = (jnp.dot(m0, wn_ref[0:128, :], preferred_element_type=jnp.float32)
          + jnp.dot(m1, wn_ref[128:256, :], preferred_element_type=jnp.float32)
          + jnp.dot(h1b, ws_ref[...], preferred_element_type=jnp.float32)
          + b_ref[...])
    av = av_ref[...]
    s0 = jnp.sum(jnp.tanh(h1b) * av, axis=1, keepdims=True)
    s1 = jnp.sum(jnp.tanh(h2) * av, axis=1, keepdims=True)
    m = jnp.maximum(s0, s1)
    e0 = jnp.exp(s0 - m)
    e1 = jnp.exp(s1 - m)
    den = e0 + e1
    a0 = e0 / den
    a1 = e1 / den
    h = a0 * h1b + a1 * h2
    o_ref[...] = (jnp.dot(h, wo_ref[...], preferred_element_type=jnp.float32)
                  + bo_ref[...])
    alpha_ref[...] = jnp.concatenate([a0, a1], axis=1)


def _edge_prep(src, dst, e_pad, trash_row, nch):
    src = src.astype(jnp.int32)
    dst = dst.astype(jnp.int32)
    pad = e_pad - src.shape[0]
    if pad:
        src = jnp.concatenate([src, jnp.zeros((pad,), jnp.int32)])
        dst = jnp.concatenate([dst, jnp.full((pad,), trash_row, jnp.int32)])
    s2 = src * 2
    gsrc = jnp.stack([s2, s2 + 1]).reshape(NC, NS, nch, CH)
    dstr = dst.reshape(NS, nch, CH)
    return gsrc, dstr


def kernel(x, edge_index_0, edge_index_1, W_n0, W_s0, b0, W_n1, W_s1, b1,
           att_vec, W_out, b_out):
    f32 = jnp.float32
    ones = jnp.ones((CH, 128), f32)
    zeros = jnp.zeros((632, 128), f32)

    _seg0, _seg1, _cntk = _seg_kernels()
    gsrc0, dstr0 = _edge_prep(edge_index_0[0], edge_index_0[1], _E0P, N1,
                              5 * IB)
    gsrc1, dstr1 = _edge_prep(edge_index_1[0], edge_index_1[1], E1, B, IB)

    cnt0, cnt1 = _cntk(dstr0, dstr1, ones, zeros)
    x2 = x.reshape(2 * N0, 128)
    x2, cnt0 = lax.optimization_barrier((x2, cnt0))
    (agg0,) = _seg0(x2, gsrc0, dstr0, zeros)

    bm0 = 1000
    h1 = pl.pallas_call(
        _l0_body,
        grid=(N1 // bm0,),
        in_specs=[
            pl.BlockSpec((NC, bm0, 128), lambda i: (0, i, 0)),
            pl.BlockSpec((NC, bm0, 128), lambda i: (0, i, 0)),
            pl.BlockSpec((bm0, F), lambda i: (i, 0)),
            pl.BlockSpec((F, HID), lambda i: (0, 0)),
            pl.BlockSpec((F, HID), lambda i: (0, 0)),
            pl.BlockSpec((1, HID), lambda i: (0, 0)),
        ],
        out_specs=pl.BlockSpec((bm0, HID), lambda i: (i, 0)),
        out_shape=jax.ShapeDtypeStruct((N1, HID), f32),
    )(agg0, cnt0, x, W_n0, W_s0, b0.reshape(1, HID))

    h1_2 = h1.reshape(2 * N1, 128)
    (agg1,) = _seg1(h1_2, gsrc1, dstr1, zeros)

    bm1 = 512
    out, alpha = pl.pallas_call(
        _l1_body,
        grid=(B // bm1,),
        in_specs=[
            pl.BlockSpec((NC, bm1, 128), lambda i: (0, i, 0)),
            pl.BlockSpec((NC, bm1, 128), lambda i: (0, i, 0)),
            pl.BlockSpec((bm1, HID), lambda i: (i, 0)),
            pl.BlockSpec((HID, HID), lambda i: (0, 0)),
            pl.BlockSpec((HID, HID), lambda i: (0, 0)),
            pl.BlockSpec((1, HID), lambda i: (0, 0)),
            pl.BlockSpec((1, HID), lambda i: (0, 0)),
            pl.BlockSpec((HID, NCLS), lambda i: (0, 0)),
            pl.BlockSpec((1, NCLS), lambda i: (0, 0)),
        ],
        out_specs=[
            pl.BlockSpec((bm1, NCLS), lambda i: (i, 0)),
            pl.BlockSpec((bm1, 2), lambda i: (i, 0)),
        ],
        out_shape=[
            jax.ShapeDtypeStruct((B, NCLS), f32),
            jax.ShapeDtypeStruct((B, 2), f32),
        ],
    )(agg1, cnt1, h1, W_n1, W_s1, b1.reshape(1, HID),
      att_vec.reshape(1, HID), W_out, b_out.reshape(1, NCLS))

    return (out, alpha)
